# baseline (speedup 1.0000x reference)
import functools

import jax
import jax.numpy as jnp
from jax import lax
from jax.experimental import pallas as pl
from jax.experimental.pallas import tpu as pltpu
from jax.experimental.pallas import tpu_sc as plsc

B, N, EMBED = 8, 1728, 768
BN = B * N
NC, NS, LANES = 2, 16, 16
NW = NC * NS
PW = BN // NW
G16 = PW // LANES
ACH = 72
NACH = PW // ACH
D, H, W = 91, 109, 91
NSCAL = 28
VPAD = 128
KDIM = 3 * VPAD
ROWS = 3456
NSTEPS = BN // ROWS


def _bf16_round(v):
    u = lax.bitcast_convert_type(v, jnp.int32)
    lsb = jnp.bitwise_and(lax.shift_right_logical(u, jnp.full((LANES,), 16, jnp.int32)),
                          jnp.full((LANES,), 1, jnp.int32))
    u = u + lsb + jnp.full((LANES,), 0x7FFF, jnp.int32)
    u = jnp.bitwise_and(u, jnp.full((LANES,), -65536, jnp.int32))
    return lax.bitcast_convert_type(u, jnp.float32)


def _sc_body(xs, ys, zs, scal, aal, out,
             x_v, y_v, z_v, scal_v, lin_v, inb_v, reg_v, rid_v, sem_a):
    wid = lax.axis_index("s") * NC + lax.axis_index("c")
    base = wid * PW

    pltpu.sync_copy(xs.at[pl.ds(base, PW)], x_v)
    pltpu.sync_copy(ys.at[pl.ds(base, PW)], y_v)
    pltpu.sync_copy(zs.at[pl.ds(base, PW)], z_v)
    pltpu.sync_copy(scal, scal_v)

    a = [scal_v[r] for r in range(16)]
    ib = [scal_v[16 + r] for r in range(12)]
    magic = jnp.full((LANES,), 12582912.0, jnp.float32)
    zero = jnp.zeros((LANES,), jnp.int32)
    one = jnp.ones((LANES,), jnp.int32)

    for g in range(G16):
        sl = pl.ds(g * LANES, LANES)
        xf = x_v[sl].astype(jnp.float32)
        yf = y_v[sl].astype(jnp.float32)
        zf = z_v[sl].astype(jnp.float32)
        w = []
        for i in range(4):
            wi = a[4 * i] * xf + a[4 * i + 1] * yf
            wi = wi + a[4 * i + 2] * zf
            wi = wi + a[4 * i + 3]
            w.append(_bf16_round(wi))
        cc = []
        for i in range(3):
            ci = ib[4 * i] * w[0] + ib[4 * i + 1] * w[1]
            ci = ci + ib[4 * i + 2] * w[2]
            ci = ci + ib[4 * i + 3] * w[3]
            cc.append((ci + magic) - magic)
        xi = cc[0].astype(jnp.int32)
        yi = cc[1].astype(jnp.int32)
        zi = cc[2].astype(jnp.int32)
        in_b = ((xi >= 0) & (xi < D) & (yi >= 0) & (yi < H)
                & (zi >= 0) & (zi < W))
        xc = jnp.minimum(jnp.maximum(xi, 0), D - 1)
        yc = jnp.minimum(jnp.maximum(yi, 0), H - 1)
        zc = jnp.minimum(jnp.maximum(zi, 0), W - 1)
        lin_v[sl] = (xc * (H * W) + yc * W) + zc
        inb_v[sl] = jnp.where(in_b, one, zero)

    acopies = [
        pltpu.async_copy(aal.at[lin_v.at[pl.ds(c * ACH, ACH)]],
                         reg_v.at[pl.ds(c * ACH, ACH)], sem_a)
        for c in range(NACH)
    ]
    for cp in acopies:
        cp.wait()

    for g in range(G16):
        sl = pl.ds(g * LANES, LANES)
        reg = reg_v[sl]
        valid = ((inb_v[sl] == one) & (reg >= zero)
                 & (reg <= jnp.full((LANES,), 116, jnp.int32)))
        rid_v[sl] = jnp.where(valid, reg, zero)

    pltpu.sync_copy(rid_v, out.at[pl.ds(base, PW)])


@jax.jit
def _launch_sc(xs, ys, zs, scal, aal_flat):
    mesh = plsc.VectorSubcoreMesh(core_axis_name="c", subcore_axis_name="s",
                                  num_cores=NC, num_subcores=NS)
    return pl.kernel(
        _sc_body,
        out_type=jax.ShapeDtypeStruct((BN,), jnp.int32),
        mesh=mesh,
        scratch_types=[
            pltpu.VMEM((PW,), jnp.int32),
            pltpu.VMEM((PW,), jnp.int32),
            pltpu.VMEM((PW,), jnp.int32),
            pltpu.VMEM((NSCAL, LANES), jnp.float32),
            pltpu.VMEM((PW,), jnp.int32),
            pltpu.VMEM((PW,), jnp.int32),
            pltpu.VMEM((PW,), jnp.int32),
            pltpu.VMEM((PW,), jnp.int32),
            pltpu.SemaphoreType.DMA,
        ],
    )(xs, ys, zs, scal, aal_flat)


def _tc_body(rid_ref, tab_ref, out_ref):
    rid = rid_ref[0, 0, :].reshape(ROWS, 1)
    cols = lax.broadcasted_iota(jnp.int32, (ROWS, VPAD), 1)
    onehot = (cols == rid).astype(jnp.bfloat16)
    t = tab_ref[...]
    t_hi = t.astype(jnp.bfloat16)
    r1 = t - t_hi.astype(jnp.float32)
    t_mid = r1.astype(jnp.bfloat16)
    t_lo = (r1 - t_mid.astype(jnp.float32)).astype(jnp.bfloat16)
    dims = (((1,), (0,)), ((), ()))
    acc = lax.dot_general(onehot, t_hi, dims,
                          preferred_element_type=jnp.float32)
    acc = acc + lax.dot_general(onehot, t_mid, dims,
                                preferred_element_type=jnp.float32)
    acc = acc + lax.dot_general(onehot, t_lo, dims,
                                preferred_element_type=jnp.float32)
    out_ref[...] = acc


@jax.jit
def _launch_tc(rid3, table_stack):
    return pl.pallas_call(
        _tc_body,
        grid=(NSTEPS,),
        in_specs=[
            pl.BlockSpec((1, 1, ROWS), lambda i: (i, 0, 0)),
            pl.BlockSpec((VPAD, EMBED), lambda i: (0, 0)),
        ],
        out_specs=pl.BlockSpec((ROWS, EMBED), lambda i: (i, 0)),
        out_shape=jax.ShapeDtypeStruct((BN, EMBED), jnp.float32),
    )(rid3, table_stack)


def kernel(patch_centers_voxels, mri_affine, aal_data, aal_affine, embed_table):
    inv4 = jnp.linalg.inv(aal_affine)
    a_bf = mri_affine.astype(jnp.bfloat16).astype(jnp.float32).reshape(16)
    i_bf = inv4.astype(jnp.bfloat16).astype(jnp.float32)[:3, :].reshape(12)
    scal = jnp.concatenate([a_bf, i_bf]).reshape(NSCAL, 1)
    scal = jnp.broadcast_to(scal, (NSCAL, LANES))
    pc = patch_centers_voxels.reshape(BN, 3)
    aal_flat = aal_data.astype(jnp.int32).reshape(D * H * W)
    rid = _launch_sc(pc[:, 0], pc[:, 1], pc[:, 2], scal, aal_flat)
    table_pad = jnp.concatenate(
        [embed_table, jnp.zeros((VPAD - embed_table.shape[0], EMBED),
                                jnp.float32)])
    out = _launch_tc(rid.reshape(NSTEPS, 1, ROWS), table_pad)
    return out.reshape(B, N, EMBED)


# speedup vs baseline: 1.0371x; 1.0371x over previous
import functools

import jax
import jax.numpy as jnp
from jax import lax
from jax.experimental import pallas as pl
from jax.experimental.pallas import tpu as pltpu
from jax.experimental.pallas import tpu_sc as plsc

B, N, EMBED = 8, 1728, 768
BN = B * N
NC, NS, LANES = 2, 16, 16
NW = NC * NS
PW = BN // NW
G16 = PW // LANES
ACH = 72
NACH = PW // ACH
D, H, W = 91, 109, 91
NSCAL = 28
VPAD = 128
KDIM = 3 * VPAD
ROWS = 3456
NSTEPS = BN // ROWS


def _bf16_round(v):
    u = lax.bitcast_convert_type(v, jnp.int32)
    lsb = jnp.bitwise_and(lax.shift_right_logical(u, jnp.full((LANES,), 16, jnp.int32)),
                          jnp.full((LANES,), 1, jnp.int32))
    u = u + lsb + jnp.full((LANES,), 0x7FFF, jnp.int32)
    u = jnp.bitwise_and(u, jnp.full((LANES,), -65536, jnp.int32))
    return lax.bitcast_convert_type(u, jnp.float32)


def _sc_body(xs, ys, zs, scal, aal, out,
             x_v, y_v, z_v, scal_v, lin_v, inb_v, reg_v, rid_v, sem_a):
    wid = lax.axis_index("s") * NC + lax.axis_index("c")
    base = wid * PW

    pltpu.sync_copy(xs.at[pl.ds(base, PW)], x_v)
    pltpu.sync_copy(ys.at[pl.ds(base, PW)], y_v)
    pltpu.sync_copy(zs.at[pl.ds(base, PW)], z_v)
    pltpu.sync_copy(scal, scal_v)

    a = [scal_v[r] for r in range(16)]
    ib = [scal_v[16 + r] for r in range(12)]
    magic = jnp.full((LANES,), 12582912.0, jnp.float32)
    zero = jnp.zeros((LANES,), jnp.int32)
    one = jnp.ones((LANES,), jnp.int32)

    for g in range(G16):
        sl = pl.ds(g * LANES, LANES)
        xf = x_v[sl].astype(jnp.float32)
        yf = y_v[sl].astype(jnp.float32)
        zf = z_v[sl].astype(jnp.float32)
        w = []
        for i in range(4):
            wi = a[4 * i] * xf + a[4 * i + 1] * yf
            wi = wi + a[4 * i + 2] * zf
            wi = wi + a[4 * i + 3]
            w.append(_bf16_round(wi))
        cc = []
        for i in range(3):
            ci = ib[4 * i] * w[0] + ib[4 * i + 1] * w[1]
            ci = ci + ib[4 * i + 2] * w[2]
            ci = ci + ib[4 * i + 3] * w[3]
            cc.append((ci + magic) - magic)
        xi = cc[0].astype(jnp.int32)
        yi = cc[1].astype(jnp.int32)
        zi = cc[2].astype(jnp.int32)
        in_b = ((xi >= 0) & (xi < D) & (yi >= 0) & (yi < H)
                & (zi >= 0) & (zi < W))
        xc = jnp.minimum(jnp.maximum(xi, 0), D - 1)
        yc = jnp.minimum(jnp.maximum(yi, 0), H - 1)
        zc = jnp.minimum(jnp.maximum(zi, 0), W - 1)
        lin_v[sl] = (xc * (H * W) + yc * W) + zc
        inb_v[sl] = jnp.where(in_b, one, zero)

    acopies = [
        pltpu.async_copy(aal.at[lin_v.at[pl.ds(c * ACH, ACH)]],
                         reg_v.at[pl.ds(c * ACH, ACH)], sem_a)
        for c in range(NACH)
    ]
    for cp in acopies:
        cp.wait()

    for g in range(G16):
        sl = pl.ds(g * LANES, LANES)
        reg = reg_v[sl]
        valid = ((inb_v[sl] == one) & (reg >= zero)
                 & (reg <= jnp.full((LANES,), 116, jnp.int32)))
        rid_v[sl] = jnp.where(valid, reg, zero)

    pltpu.sync_copy(rid_v, out.at[pl.ds(base, PW)])


@jax.jit
def _launch_sc(xs, ys, zs, scal, aal_flat):
    mesh = plsc.VectorSubcoreMesh(core_axis_name="c", subcore_axis_name="s",
                                  num_cores=NC, num_subcores=NS)
    return pl.kernel(
        _sc_body,
        out_type=jax.ShapeDtypeStruct((BN,), jnp.int32),
        mesh=mesh,
        scratch_types=[
            pltpu.VMEM((PW,), jnp.int32),
            pltpu.VMEM((PW,), jnp.int32),
            pltpu.VMEM((PW,), jnp.int32),
            pltpu.VMEM((NSCAL, LANES), jnp.float32),
            pltpu.VMEM((PW,), jnp.int32),
            pltpu.VMEM((PW,), jnp.int32),
            pltpu.VMEM((PW,), jnp.int32),
            pltpu.VMEM((PW,), jnp.int32),
            pltpu.SemaphoreType.DMA,
        ],
    )(xs, ys, zs, scal, aal_flat)


def _tc_body(rid_ref, tab_ref, out_ref):
    rid = rid_ref[0, 0, :].reshape(ROWS, 1)
    cols = lax.broadcasted_iota(jnp.int32, (ROWS, KDIM), 1)
    onehot3 = ((cols & (VPAD - 1)) == rid).astype(jnp.bfloat16)
    t = tab_ref[...]
    t_hi = t.astype(jnp.bfloat16)
    r1 = t - t_hi.astype(jnp.float32)
    t_mid = r1.astype(jnp.bfloat16)
    t_lo = (r1 - t_mid.astype(jnp.float32)).astype(jnp.bfloat16)
    stacked = jnp.concatenate([t_hi, t_mid, t_lo], axis=0)
    out_ref[...] = lax.dot_general(onehot3, stacked,
                                   (((1,), (0,)), ((), ())),
                                   preferred_element_type=jnp.float32)


@jax.jit
def _launch_tc(rid3, table_stack):
    return pl.pallas_call(
        _tc_body,
        grid=(NSTEPS,),
        in_specs=[
            pl.BlockSpec((1, 1, ROWS), lambda i: (i, 0, 0)),
            pl.BlockSpec((VPAD, EMBED), lambda i: (0, 0)),
        ],
        out_specs=pl.BlockSpec((ROWS, EMBED), lambda i: (i, 0)),
        out_shape=jax.ShapeDtypeStruct((BN, EMBED), jnp.float32),
    )(rid3, table_stack)


def kernel(patch_centers_voxels, mri_affine, aal_data, aal_affine, embed_table):
    inv4 = jnp.linalg.inv(aal_affine)
    a_bf = mri_affine.astype(jnp.bfloat16).astype(jnp.float32).reshape(16)
    i_bf = inv4.astype(jnp.bfloat16).astype(jnp.float32)[:3, :].reshape(12)
    scal = jnp.concatenate([a_bf, i_bf]).reshape(NSCAL, 1)
    scal = jnp.broadcast_to(scal, (NSCAL, LANES))
    pc = patch_centers_voxels.reshape(BN, 3)
    aal_flat = aal_data.astype(jnp.int32).reshape(D * H * W)
    rid = _launch_sc(pc[:, 0], pc[:, 1], pc[:, 2], scal, aal_flat)
    table_pad = jnp.concatenate(
        [embed_table, jnp.zeros((VPAD - embed_table.shape[0], EMBED),
                                jnp.float32)])
    out = _launch_tc(rid.reshape(NSTEPS, 1, ROWS), table_pad)
    return out.reshape(B, N, EMBED)
